# baseline (speedup 1.0000x reference)
import functools
import jax
import jax.numpy as jnp
from jax import lax
from jax.experimental import pallas as pl
from jax.experimental.pallas import tpu as pltpu
from jax.experimental.pallas import tpu_sc as plsc


def _lane_perm(v, perm_idx):
    return lax.gather(
        v,
        perm_idx[:, None],
        lax.GatherDimensionNumbers(
            offset_dims=(), collapsed_slice_dims=(0,), start_index_map=(0,)
        ),
        slice_sizes=(1,),
        mode=lax.GatherScatterMode.PROMISE_IN_BOUNDS,
    )


def kernel(batch_mask, mask_emb):
    M, N = batch_mask.shape
    _, D = mask_emb.shape
    B = M * N
    NC, NS, L = 2, 16, 16
    NW = NC * NS
    P = B // 2
    p_per_w = P // NW
    PCH = 256
    ICH = 2 * PCH
    n_ch = p_per_w // PCH

    idx = batch_mask.reshape(B)

    combo = jnp.concatenate(
        [
            jnp.concatenate([mask_emb[c >> 1], mask_emb[c & 1]])[None, :]
            for c in range(4)
        ],
        axis=0,
    )

    mesh = plsc.VectorSubcoreMesh(
        core_axis_name="c", subcore_axis_name="s", num_cores=NC, num_subcores=NS
    )

    @functools.partial(
        pl.kernel,
        mesh=mesh,
        out_type=jax.ShapeDtypeStruct((P, 2 * D), jnp.float32),
        scratch_types=[
            pltpu.VMEM((ICH,), jnp.int32),
            pltpu.VMEM((PCH, 2 * D), jnp.float32),
            pltpu.SemaphoreType.DMA,
        ],
    )
    def k(combo_hbm, idx_hbm, out_hbm, idx_v, rows_v, sem):
        wid = lax.axis_index("s") * NC + lax.axis_index("c")
        ibase = wid * 2 * p_per_w
        obase = wid * p_per_w

        ii = lax.iota(jnp.int32, L)
        rot1 = (ii + 1) % L
        even2 = (2 * ii) % L
        lo8 = ii < 8

        def step(i, carry):
            pltpu.sync_copy(idx_hbm.at[pl.ds(ibase + i * ICH, ICH)], idx_v)
            copies = []
            for q in range(PCH // L):
                w0 = idx_v[pl.ds(2 * L * q, L)]
                w1 = idx_v[pl.ds(2 * L * q + L, L)]
                cc0 = 2 * w0 + _lane_perm(w0, rot1)
                cc1 = 2 * w1 + _lane_perm(w1, rot1)
                z = jnp.where(lo8, _lane_perm(cc0, even2), _lane_perm(cc1, even2))
                copies.append(
                    pltpu.async_copy(
                        combo_hbm.at[z], rows_v.at[pl.ds(L * q, L)], sem
                    )
                )
            for cp in copies:
                cp.wait()
            pltpu.sync_copy(rows_v, out_hbm.at[pl.ds(obase + i * PCH, PCH)])
            return carry

        lax.fori_loop(0, n_ch, step, 0)

    out = k(combo, idx)
    return out.reshape(M, N, D)
